# Initial kernel scaffold; baseline (speedup 1.0000x reference)
import jax
import jax.numpy as jnp
from jax.experimental import pallas as pl


def kernel(x, pe):
    raise NotImplementedError("write your pallas kernel here")


# speedup vs baseline: 1.9909x; 1.9909x over previous
import functools

import jax
import jax.numpy as jnp
from jax import lax
from jax.experimental import pallas as pl
from jax.experimental.pallas import tpu as pltpu
from jax.experimental.pallas import tpu_sc as plsc

D_MODEL = 1024
NUM_CORES = 2
NUM_SUBCORES = 16
NW = NUM_CORES * NUM_SUBCORES
CHUNK = 32


@functools.cache
def _make_gather(B, D):
    b_per_w = B // NW
    nchunk = b_per_w // CHUNK
    mesh = plsc.VectorSubcoreMesh(
        core_axis_name="c", subcore_axis_name="s",
        num_cores=NUM_CORES, num_subcores=NUM_SUBCORES)

    @functools.partial(
        pl.kernel, mesh=mesh,
        out_type=jax.ShapeDtypeStruct((B, D), jnp.float32),
        scratch_types=[
            pltpu.VMEM((b_per_w,), jnp.int32),
            pltpu.VMEM((CHUNK, D), jnp.float32),
            pltpu.SemaphoreType.DMA,
        ],
    )
    def k(idx_hbm, pe_hbm, out_hbm, idx_v, rows_v, sem):
        wid = lax.axis_index("s") * NUM_CORES + lax.axis_index("c")
        base = wid * b_per_w
        pltpu.sync_copy(idx_hbm.at[pl.ds(base, b_per_w)], idx_v)

        def body(c, carry):
            off = c * CHUNK
            pltpu.async_copy(
                pe_hbm.at[idx_v.at[pl.ds(off, CHUNK)]], rows_v, sem).wait()
            pltpu.sync_copy(rows_v, out_hbm.at[pl.ds(base + off, CHUNK)])
            return carry

        lax.fori_loop(0, nchunk, body, 0)

    return k


def kernel(x, pe):
    idx = x.reshape(-1)
    return _make_gather(idx.shape[0], pe.shape[1])(idx, pe)


# speedup vs baseline: 2.3114x; 1.1610x over previous
import functools

import jax
import jax.numpy as jnp
from jax import lax
from jax.experimental import pallas as pl
from jax.experimental.pallas import tpu as pltpu
from jax.experimental.pallas import tpu_sc as plsc

D_MODEL = 1024
NUM_CORES = 2
NUM_SUBCORES = 16
NW = NUM_CORES * NUM_SUBCORES
CHUNK = 16
NBUF = 4


@functools.cache
def _make_gather(B, D):
    b_per_w = B // NW
    nchunk = b_per_w // CHUNK
    assert nchunk % NBUF == 0
    mesh = plsc.VectorSubcoreMesh(
        core_axis_name="c", subcore_axis_name="s",
        num_cores=NUM_CORES, num_subcores=NUM_SUBCORES)

    @functools.partial(
        pl.kernel, mesh=mesh,
        out_type=jax.ShapeDtypeStruct((B, D), jnp.float32),
        scratch_types=[
            pltpu.VMEM((b_per_w,), jnp.int32),
            pltpu.VMEM((NBUF, CHUNK, D), jnp.float32),
        ] + [pltpu.SemaphoreType.DMA] * (2 * NBUF),
    )
    def k(idx_hbm, pe_hbm, out_hbm, idx_v, bufs, *sems):
        gsems, ssems = sems[:NBUF], sems[NBUF:]
        wid = lax.axis_index("s") * NUM_CORES + lax.axis_index("c")
        base = wid * b_per_w
        pltpu.sync_copy(idx_hbm.at[pl.ds(base, b_per_w)], idx_v)

        def gather_start(c, b):
            pltpu.async_copy(
                pe_hbm.at[idx_v.at[pl.ds(c * CHUNK, CHUNK)]],
                bufs.at[b], gsems[b])

        def gather_wait(b):
            pltpu.make_async_copy(
                pe_hbm.at[pl.ds(0, CHUNK)], bufs.at[b], gsems[b]).wait()

        def write_start(c, b):
            pltpu.async_copy(
                bufs.at[b], out_hbm.at[pl.ds(base + c * CHUNK, CHUNK)],
                ssems[b])

        def write_wait(b):
            pltpu.make_async_copy(
                bufs.at[b], out_hbm.at[pl.ds(0, CHUNK)], ssems[b]).wait()

        for b in range(NBUF):
            gather_start(b, b)

        def outer(g, carry):
            for b in range(NBUF):
                gather_wait(b)
                write_start(g + b, b)
            for b in range(NBUF):
                nxt = g + b + NBUF

                @pl.when(nxt < nchunk)
                def _():
                    write_wait(b)
                    gather_start(nxt, b)
            return carry

        lax.fori_loop(0, nchunk // NBUF, lambda i, c: outer(i * NBUF, c), 0)
        for b in range(NBUF):
            write_wait(b)

    return k


def kernel(x, pe):
    idx = x.reshape(-1)
    return _make_gather(idx.shape[0], pe.shape[1])(idx, pe)
